# baseline (speedup 1.0000x reference)
import jax
import jax.numpy as jnp
from jax.experimental import pallas as pl
from jax.experimental.pallas import tpu as pltpu

_B, _Wdim, _F = 16, 256, 64
_H, _D = 2, 256
_ALPHA = 0.2
_NB = 16
_NCOLS = 3 * _H * _D + 128


def _gat_batch_kernel(xt_ref, wt_ref, bb_ref, sgw_ref, o_ref, pa_ref, pb_ref,
                      e_ref):
    for bb in range(_NB):
        xbt = jnp.transpose(xt_ref[bb]).astype(jnp.bfloat16)
        pa_ref[bb] = (jnp.dot(xbt, wt_ref[...],
                              preferred_element_type=jnp.float32)
                      + bb_ref[...])
        pb_ref[bb] = pa_ref[bb, :, 0:2 * _H * _D].astype(jnp.bfloat16)

    def emit_logits(bb, h):
        sth = pb_ref[bb, :, h * _D:(h + 1) * _D]
        sgc = sgw_ref[:, h:h + 1]
        for j in range(_F):
            ttrow = pb_ref[bb, j, 512 + h * _D:512 + (h + 1) * _D][None, :]
            w = jnp.abs(sth + ttrow)
            e_ref[bb, h * _F + j:h * _F + j + 1, :] = jax.lax.dot_general(
                sgc, w, (((0,), (1,)), ((), ())),
                preferred_element_type=jnp.float32)

    def emit_tail(bb, h):
        slin = pa_ref[bb, :, 1536 + h:1537 + h]
        e_t = e_ref[bb, h * _F:(h + 1) * _F, :] + jnp.transpose(slin)
        m = jnp.max(e_t, axis=1, keepdims=True)
        ex = jnp.exp(e_t - m)
        a_t = ex / jnp.sum(ex, axis=1, keepdims=True)
        srawT = pa_ref[bb, :, 1024 + h * _D:1024 + (h + 1) * _D]
        return jax.lax.dot_general(
            srawT.astype(jnp.bfloat16), a_t.astype(jnp.bfloat16),
            (((0,), (1,)), ((), ())),
            preferred_element_type=jnp.float32)

    units = [(bb, h) for bb in range(_NB) for h in range(_H)]
    outs = {}
    for k, (bb, h) in enumerate(units):
        emit_logits(bb, h)
        if k > 0:
            pbb, ph = units[k - 1]
            outs[(pbb, ph)] = emit_tail(pbb, ph)
            if ph == _H - 1:
                o_ref[pbb] = 0.5 * (outs[(pbb, 0)] + outs[(pbb, 1)])
    lbb, lh = units[-1]
    outs[(lbb, lh)] = emit_tail(lbb, lh)
    o_ref[lbb] = 0.5 * (outs[(lbb, 0)] + outs[(lbb, 1)])


def kernel(x, W_src, b_src, W_dst, b_dst, attn):
    af = attn.reshape(_H * _D)
    sc = 0.4 * jnp.abs(af)
    wlin = jnp.stack([
        W_src[:, h * _D:(h + 1) * _D] @ (0.6 * attn[h]) for h in range(_H)
    ], axis=1)
    blin = jnp.stack([
        (0.6 * attn[h]) @ b_src[h * _D:(h + 1) * _D] for h in range(_H)
    ])
    wt = jnp.concatenate([
        W_src * sc[None, :], W_dst * sc[None, :], W_src, wlin,
        jnp.zeros((_Wdim, 126), jnp.float32),
    ], axis=1)
    bb = jnp.concatenate([
        b_src * sc, b_dst * sc, b_src, blin, jnp.zeros((126,), jnp.float32),
    ])[None, :]
    sgw = jnp.sign(attn).T.astype(jnp.bfloat16)
    wt = wt.astype(jnp.bfloat16)

    grid = (_B // _NB,)
    out = pl.pallas_call(
        _gat_batch_kernel,
        grid=grid,
        in_specs=[
            pl.BlockSpec((_NB, _Wdim, _F), lambda b: (b, 0, 0)),
            pl.BlockSpec((_Wdim, _NCOLS), lambda b: (0, 0)),
            pl.BlockSpec((1, _NCOLS), lambda b: (0, 0)),
            pl.BlockSpec((_D, _H), lambda b: (0, 0)),
        ],
        out_specs=pl.BlockSpec((_NB, _D, _F), lambda b: (b, 0, 0)),
        out_shape=jax.ShapeDtypeStruct((_B, _D, _F), jnp.float32),
        scratch_shapes=[
            pltpu.VMEM((_NB, _F, _NCOLS), jnp.float32),
            pltpu.VMEM((_NB, _F, 2 * _H * _D), jnp.bfloat16),
            pltpu.VMEM((_NB, _H * _F, _F), jnp.float32),
        ],
        compiler_params=pltpu.CompilerParams(
            dimension_semantics=("parallel",),
        ),
    )(x, wt, bb, sgw)
    return out


# speedup vs baseline: 1.2315x; 1.2315x over previous
import jax
import jax.numpy as jnp
from jax.experimental import pallas as pl
from jax.experimental.pallas import tpu as pltpu

_B, _Wdim, _F = 16, 256, 64
_H, _D = 2, 256
_ALPHA = 0.2


def _gat_kernel(x_ref, ws_ref, wd_ref, bs_ref, bd_ref, attn_ref,
                o_ref, pa_ref, pb_ref, e_ref, sl_ref):
    af = attn_ref[...]
    sc04 = 0.4 * jnp.abs(jnp.reshape(af, (1, _H * _D)))
    at = jnp.transpose(af)
    a06 = 0.6 * at
    sgw = jnp.sign(at).astype(jnp.bfloat16)

    for bb in range(_B):
        xbt = jnp.transpose(x_ref[bb]).astype(jnp.bfloat16)
        ps = (jnp.dot(xbt, ws_ref[...], preferred_element_type=jnp.float32)
              + bs_ref[...])
        pa_ref[bb] = ps
        pb_ref[bb, :, 0:_H * _D] = (ps * sc04).astype(jnp.bfloat16)
        pd = (jnp.dot(xbt, wd_ref[...], preferred_element_type=jnp.float32)
              + bd_ref[...])
        pb_ref[bb, :, _H * _D:2 * _H * _D] = (pd * sc04).astype(jnp.bfloat16)
        for h in range(_H):
            sl_ref[bb, :, h:h + 1] = jnp.dot(
                ps[:, h * _D:(h + 1) * _D], a06[:, h:h + 1],
                preferred_element_type=jnp.float32)

    def emit_logits(bb, h):
        sth = pb_ref[bb, :, h * _D:(h + 1) * _D]
        sgc = sgw[:, h:h + 1]
        for j in range(_F):
            ttrow = pb_ref[bb, j, 512 + h * _D:512 + (h + 1) * _D][None, :]
            w = jnp.abs(sth + ttrow)
            e_ref[bb, h * _F + j:h * _F + j + 1, :] = jax.lax.dot_general(
                sgc, w, (((0,), (1,)), ((), ())),
                preferred_element_type=jnp.float32)

    def emit_tail(bb, h):
        slin = sl_ref[bb, :, h:h + 1]
        e_t = e_ref[bb, h * _F:(h + 1) * _F, :] + jnp.transpose(slin)
        m = jnp.max(e_t, axis=1, keepdims=True)
        ex = jnp.exp(e_t - m)
        a_t = ex / jnp.sum(ex, axis=1, keepdims=True)
        srawT = pa_ref[bb, :, h * _D:(h + 1) * _D]
        return jax.lax.dot_general(
            srawT.astype(jnp.bfloat16), a_t.astype(jnp.bfloat16),
            (((0,), (1,)), ((), ())),
            preferred_element_type=jnp.float32)

    units = [(bb, h) for bb in range(_B) for h in range(_H)]
    outs = {}
    for k, (bb, h) in enumerate(units):
        emit_logits(bb, h)
        if k > 0:
            pbb, ph = units[k - 1]
            outs[(pbb, ph)] = emit_tail(pbb, ph)
            if ph == _H - 1:
                o_ref[pbb] = 0.5 * (outs[(pbb, 0)] + outs[(pbb, 1)])
    lbb, lh = units[-1]
    outs[(lbb, lh)] = emit_tail(lbb, lh)
    o_ref[lbb] = 0.5 * (outs[(lbb, 0)] + outs[(lbb, 1)])


def kernel(x, W_src, b_src, W_dst, b_dst, attn):
    out = pl.pallas_call(
        _gat_kernel,
        grid=(1,),
        in_specs=[
            pl.BlockSpec((_B, _Wdim, _F), lambda b: (0, 0, 0)),
            pl.BlockSpec((_Wdim, _H * _D), lambda b: (0, 0)),
            pl.BlockSpec((_Wdim, _H * _D), lambda b: (0, 0)),
            pl.BlockSpec((1, _H * _D), lambda b: (0, 0)),
            pl.BlockSpec((1, _H * _D), lambda b: (0, 0)),
            pl.BlockSpec((_H, _D), lambda b: (0, 0)),
        ],
        out_specs=pl.BlockSpec((_B, _D, _F), lambda b: (0, 0, 0)),
        out_shape=jax.ShapeDtypeStruct((_B, _D, _F), jnp.float32),
        scratch_shapes=[
            pltpu.VMEM((_B, _F, _H * _D), jnp.float32),
            pltpu.VMEM((_B, _F, 2 * _H * _D), jnp.bfloat16),
            pltpu.VMEM((_B, _H * _F, _F), jnp.float32),
            pltpu.VMEM((_B, _F, _H), jnp.float32),
        ],
    )(x, W_src.astype(jnp.bfloat16), W_dst.astype(jnp.bfloat16),
      b_src[None, :], b_dst[None, :], attn)
    return out


# speedup vs baseline: 1.3253x; 1.0761x over previous
import jax
import jax.numpy as jnp
from jax.experimental import pallas as pl
from jax.experimental.pallas import tpu as pltpu

_B, _Wdim, _F = 16, 256, 64
_H, _D = 2, 256
_ALPHA = 0.2


def _gat_kernel(x_ref, ws_ref, wd_ref, bs_ref, bd_ref, attn_ref,
                o_ref, pa_ref, pb_ref, e_ref, sl_ref, wsb_ref, wdb_ref):
    af = attn_ref[...]
    sc04 = 0.4 * jnp.abs(jnp.reshape(af, (1, _H * _D)))
    at = jnp.transpose(af)
    a06 = 0.6 * at
    sgw = jnp.sign(at).astype(jnp.bfloat16)
    wsb_ref[...] = ws_ref[...].astype(jnp.bfloat16)
    wdb_ref[...] = wd_ref[...].astype(jnp.bfloat16)
    bs = bs_ref[...][None, :]
    bd = bd_ref[...][None, :]

    for bb in range(_B):
        xbt = jnp.transpose(x_ref[bb]).astype(jnp.bfloat16)
        ps = (jnp.dot(xbt, wsb_ref[...], preferred_element_type=jnp.float32)
              + bs)
        pa_ref[bb] = ps
        pb_ref[bb, :, 0:_H * _D] = (ps * sc04).astype(jnp.bfloat16)
        pd = (jnp.dot(xbt, wdb_ref[...], preferred_element_type=jnp.float32)
              + bd)
        pb_ref[bb, :, _H * _D:2 * _H * _D] = (pd * sc04).astype(jnp.bfloat16)
        for h in range(_H):
            sl_ref[bb, :, h:h + 1] = jnp.dot(
                ps[:, h * _D:(h + 1) * _D], a06[:, h:h + 1],
                preferred_element_type=jnp.float32)

    def emit_logits(bb, h):
        sth = pb_ref[bb, :, h * _D:(h + 1) * _D]
        sgc = sgw[:, h:h + 1]
        for j in range(_F):
            ttrow = pb_ref[bb, j, 512 + h * _D:512 + (h + 1) * _D][None, :]
            w = jnp.abs(sth + ttrow)
            e_ref[bb, h * _F + j:h * _F + j + 1, :] = jax.lax.dot_general(
                sgc, w, (((0,), (1,)), ((), ())),
                preferred_element_type=jnp.float32)

    def emit_tail(bb, h):
        slin = sl_ref[bb, :, h:h + 1]
        e_t = e_ref[bb, h * _F:(h + 1) * _F, :] + jnp.transpose(slin)
        m = jnp.max(e_t, axis=1, keepdims=True)
        ex = jnp.exp(e_t - m)
        a_t = ex / jnp.sum(ex, axis=1, keepdims=True)
        srawT = pa_ref[bb, :, h * _D:(h + 1) * _D]
        return jax.lax.dot_general(
            srawT.astype(jnp.bfloat16), a_t.astype(jnp.bfloat16),
            (((0,), (1,)), ((), ())),
            preferred_element_type=jnp.float32)

    units = [(bb, h) for bb in range(_B) for h in range(_H)]
    outs = {}
    for k, (bb, h) in enumerate(units):
        emit_logits(bb, h)
        if k > 0:
            pbb, ph = units[k - 1]
            outs[(pbb, ph)] = emit_tail(pbb, ph)
            if ph == _H - 1:
                o_ref[pbb] = 0.5 * (outs[(pbb, 0)] + outs[(pbb, 1)])
    lbb, lh = units[-1]
    outs[(lbb, lh)] = emit_tail(lbb, lh)
    o_ref[lbb] = 0.5 * (outs[(lbb, 0)] + outs[(lbb, 1)])


def kernel(x, W_src, b_src, W_dst, b_dst, attn):
    out = pl.pallas_call(
        _gat_kernel,
        grid=(1,),
        in_specs=[
            pl.BlockSpec((_B, _Wdim, _F), lambda b: (0, 0, 0)),
            pl.BlockSpec((_Wdim, _H * _D), lambda b: (0, 0)),
            pl.BlockSpec((_Wdim, _H * _D), lambda b: (0, 0)),
            pl.BlockSpec((_H * _D,), lambda b: (0,)),
            pl.BlockSpec((_H * _D,), lambda b: (0,)),
            pl.BlockSpec((_H, _D), lambda b: (0, 0)),
        ],
        out_specs=pl.BlockSpec((_B, _D, _F), lambda b: (0, 0, 0)),
        out_shape=jax.ShapeDtypeStruct((_B, _D, _F), jnp.float32),
        scratch_shapes=[
            pltpu.VMEM((_B, _F, _H * _D), jnp.float32),
            pltpu.VMEM((_B, _F, 2 * _H * _D), jnp.bfloat16),
            pltpu.VMEM((_B, _H * _F, _F), jnp.float32),
            pltpu.VMEM((_B, _F, _H), jnp.float32),
            pltpu.VMEM((_Wdim, _H * _D), jnp.bfloat16),
            pltpu.VMEM((_Wdim, _H * _D), jnp.bfloat16),
        ],
    )(x, W_src, W_dst, b_src, b_dst, attn)
    return out
